# baseline (speedup 1.0000x reference)
import functools
import math

import jax
import jax.numpy as jnp
from jax import lax
from jax.experimental import pallas as pl
from jax.experimental.pallas import tpu as pltpu
from jax.experimental.pallas import tpu_sc as plsc

D_MODEL = 64
SCALE = math.sqrt(D_MODEL)

NUM_CORES = 2
NUM_SUBCORES = 16
NUM_WORKERS = NUM_CORES * NUM_SUBCORES

BATCH = 16384 * 50
B_PER_W = BATCH // NUM_WORKERS
CHUNK = 1024
NUM_CHUNKS = B_PER_W // CHUNK
ROW_UNROLL = 8

_mesh = plsc.VectorSubcoreMesh(core_axis_name="c", subcore_axis_name="s")


@functools.partial(
    pl.kernel,
    mesh=_mesh,
    compiler_params=pltpu.CompilerParams(use_tc_tiling_on_sc=False),
    out_type=jax.ShapeDtypeStruct((BATCH, D_MODEL), jnp.float32),
    scratch_types=[
        pltpu.VMEM((CHUNK,), jnp.int32),
        pltpu.VMEM((CHUNK, D_MODEL), jnp.float32),
        pltpu.SemaphoreType.DMA,
    ],
)
def _emb_lookup(x_hbm, table_hbm, out_hbm, idx_v, rows_v, sem):
    wid = lax.axis_index("s") * NUM_CORES + lax.axis_index("c")
    base = wid * B_PER_W

    def chunk_body(ci, carry):
        off = base + ci * CHUNK
        pltpu.sync_copy(x_hbm.at[pl.ds(off, CHUNK)], idx_v)
        pltpu.async_copy(table_hbm.at[idx_v], rows_v, sem).wait()

        def scale_body(it, c2):
            i = it * ROW_UNROLL
            for u in range(ROW_UNROLL):
                for j in range(D_MODEL // 16):
                    s = pl.ds(j * 16, 16)
                    rows_v[i + u, s] = rows_v[i + u, s] * SCALE
            return c2

        lax.fori_loop(0, CHUNK // ROW_UNROLL, scale_body, 0)
        pltpu.sync_copy(rows_v, out_hbm.at[pl.ds(off, CHUNK)])
        return carry

    lax.fori_loop(0, NUM_CHUNKS, chunk_body, 0)


def kernel(x, table):
    flat = x.reshape(-1)
    out = _emb_lookup(flat, table)
    return out.reshape(x.shape[0], x.shape[1], D_MODEL)


# speedup vs baseline: 1.0603x; 1.0603x over previous
import functools
import math

import jax
import jax.numpy as jnp
from jax import lax
from jax.experimental import pallas as pl
from jax.experimental.pallas import tpu as pltpu
from jax.experimental.pallas import tpu_sc as plsc

D_MODEL = 64
SCALE = math.sqrt(D_MODEL)

NUM_CORES = 2
NUM_SUBCORES = 16
NUM_WORKERS = NUM_CORES * NUM_SUBCORES

BATCH = 16384 * 50
B_PER_W = BATCH // NUM_WORKERS
CHUNK = 400
NUM_CHUNKS = B_PER_W // CHUNK
NBUF = 4
LOOKAHEAD = 2
ROW_UNROLL = 8

_mesh = plsc.VectorSubcoreMesh(core_axis_name="c", subcore_axis_name="s")


@functools.partial(
    pl.kernel,
    mesh=_mesh,
    compiler_params=pltpu.CompilerParams(use_tc_tiling_on_sc=False),
    out_type=jax.ShapeDtypeStruct((BATCH, D_MODEL), jnp.float32),
    scratch_types=[
        pltpu.VMEM((B_PER_W,), jnp.int32),
        pltpu.VMEM((NBUF, CHUNK, D_MODEL), jnp.float32),
        pltpu.SemaphoreType.DMA((NBUF,)),
        pltpu.SemaphoreType.DMA((NBUF,)),
    ],
)
def _emb_lookup(x_hbm, table_hbm, out_hbm, idx_v, rows_v, gsem, wsem):
    wid = lax.axis_index("s") * NUM_CORES + lax.axis_index("c")
    base = wid * B_PER_W
    pltpu.sync_copy(x_hbm.at[pl.ds(base, B_PER_W)], idx_v)

    def issue_gather(c, slot):
        pltpu.async_copy(
            table_hbm.at[idx_v.at[pl.ds(c * CHUNK, CHUNK)]],
            rows_v.at[slot],
            gsem.at[slot],
        )

    def wait_gather(slot):
        pltpu.make_async_copy(
            out_hbm.at[pl.ds(0, CHUNK)], rows_v.at[slot], gsem.at[slot]
        ).wait()

    def issue_wb(c, slot):
        pltpu.async_copy(
            rows_v.at[slot], out_hbm.at[pl.ds(base + c * CHUNK, CHUNK)], wsem.at[slot]
        )

    def wait_wb(slot):
        pltpu.make_async_copy(
            rows_v.at[slot], out_hbm.at[pl.ds(0, CHUNK)], wsem.at[slot]
        ).wait()

    for c in range(LOOKAHEAD):
        issue_gather(c, c % NBUF)

    def outer(g, carry):
        for k in range(NBUF):
            c = g * NBUF + k
            slot = k
            nslot = (k + LOOKAHEAD) % NBUF
            wait_gather(slot)

            if k < 2:
                @pl.when(g > 0)
                def _():
                    wait_wb(nslot)

                issue_gather(c + LOOKAHEAD, nslot)
            else:

                @pl.when(g < (NUM_CHUNKS // NBUF) - 1)
                def _():
                    wait_wb(nslot)
                    issue_gather(c + LOOKAHEAD, nslot)

            def scale_body(it, c2):
                i = it * ROW_UNROLL
                for u in range(ROW_UNROLL):
                    for j in range(D_MODEL // 16):
                        s = pl.ds(j * 16, 16)
                        rows_v[slot, i + u, s] = rows_v[slot, i + u, s] * SCALE
                return c2

            lax.fori_loop(0, CHUNK // ROW_UNROLL, scale_body, 0)
            issue_wb(c, slot)
        return carry

    lax.fori_loop(0, NUM_CHUNKS // NBUF, outer, 0)

    for slot in range(NBUF):
        wait_wb(slot)


def kernel(x, table):
    flat = x.reshape(-1)
    out = _emb_lookup(flat, table)
    return out.reshape(x.shape[0], x.shape[1], D_MODEL)
